# baseline (speedup 1.0000x reference)
import functools

import jax
import jax.numpy as jnp
from jax import lax
from jax.experimental import pallas as pl
from jax.experimental.pallas import tpu as pltpu
from jax.experimental.pallas import tpu_sc as plsc

_B, _CB, _R, _NA, _D = 32, 32, 64, 8, 4096
_AR = _NA * _R
_DT = 2048
_NP = _NA // 2


@functools.cache
def _make_sc_count():
    mesh = plsc.VectorSubcoreMesh(core_axis_name="c", subcore_axis_name="s", num_cores=1)
    return functools.partial(
        pl.kernel,
        out_type=jax.ShapeDtypeStruct((_B, _AR), jnp.float32),
        mesh=mesh,
        scratch_types=[
            pltpu.VMEM((_CB * _R,), jnp.int32),
            pltpu.VMEM((_CB,), jnp.int32),
            pltpu.VMEM((_B, _AR), jnp.float32),
        ],
        compiler_params=pltpu.CompilerParams(needs_layout_passes=False),
    )(_sc_count_body)


def _sc_count_body(xids_hbm, wids_hbm, zeros_hbm, out_hbm, xids_v, wids_v, c_v):
    cid = lax.axis_index("c")
    sid = lax.axis_index("s")

    @pl.when(jnp.logical_and(cid == 0, sid == 0))
    def _():
        pltpu.sync_copy(xids_hbm, xids_v)
        pltpu.sync_copy(wids_hbm, wids_v)
        pltpu.sync_copy(zeros_hbm, c_v)
        lane = lax.broadcasted_iota(jnp.int32, (16,), 0)
        ones = jnp.ones((16,), jnp.float32)

        def body(i, carry):
            b = xids_v[pl.ds(i * 16, 16)]
            c = lax.shift_right_logical(i, 2)
            a = plsc.load_gather(wids_v, [jnp.full((16,), c, jnp.int32)])
            col = a * _R + (i & 3) * 16 + lane
            plsc.addupdate_scatter(c_v, [b, col], ones)
            return carry

        lax.fori_loop(0, (_CB * _R) // 16, body, 0)
        pltpu.sync_copy(c_v, out_hbm)


def _mm1_body(xe_ref, xo_ref, a_ref, m_ref):
    for t in range(2):
        p = jnp.dot(xe_ref[...], a_ref[t], preferred_element_type=jnp.float32)
        q = jnp.dot(xo_ref[...], a_ref[t], preferred_element_type=jnp.float32)
        m_ref[t, :_B] = p
        m_ref[t, _B:] = q


def _stage1(xe, xo, a2):
    return pl.pallas_call(
        _mm1_body,
        grid=(_NA // 2,),
        in_specs=[
            pl.BlockSpec((_B, _D // 2), lambda p: (0, 0)),
            pl.BlockSpec((_B, _D // 2), lambda p: (0, 0)),
            pl.BlockSpec((2, _D // 2, 2 * _R), lambda p: (p, 0, 0)),
        ],
        out_specs=pl.BlockSpec((2, 2 * _B, 2 * _R), lambda p: (p, 0, 0)),
        out_shape=jax.ShapeDtypeStruct((_NA, 2 * _B, 2 * _R), jnp.float32),
    )(xe, xo, a2)


def _mm2_body(wids_ref, m_ref, c_ref, xids_ref, b3_ref, out_ref, w_scr):
    j = pl.program_id(0)

    @pl.when(j == 0)
    def _():
        gs = []
        for a in range(_NA):
            wa = wids_ref[a]
            mrow = m_ref[pl.ds(wa, 1)][0]
            brow = xids_ref[pl.ds(a * _R, _R)]
            msk = brow[None, :] == lax.broadcasted_iota(
                jnp.int32, (_B, _R), 0)
            sel = jnp.where(msk, 1.0, 0.0)
            sel2 = jnp.concatenate([sel, sel], axis=1)
            redp = jnp.sum(mrow[:_B] * sel2, axis=0, keepdims=True)
            redq = jnp.sum(mrow[_B:] * sel2, axis=0, keepdims=True)
            gs.append(redp[:, :_R] + redq[:, _R:])
        gfull = jnp.concatenate(gs, axis=1)
        w_scr[...] = (c_ref[...] * gfull).astype(jnp.bfloat16)

    out_ref[...] = (2.0 * jnp.dot(
        w_scr[...], b3_ref[...],
        preferred_element_type=jnp.float32)).astype(jnp.bfloat16)


def _stage2(wids, M, C2d, xids, b2):
    return pl.pallas_call(
        _mm2_body,
        grid=(_D // _DT,),
        in_specs=[
            pl.BlockSpec(memory_space=pltpu.SMEM),
            pl.BlockSpec((_NA, 2 * _B, 2 * _R), lambda j: (0, 0, 0)),
            pl.BlockSpec((_B, _AR), lambda j: (0, 0)),
            pl.BlockSpec((_AR,), lambda j: (0,)),
            pl.BlockSpec((_AR, _DT), lambda j: (0, j)),
        ],
        out_specs=pl.BlockSpec((_B, _DT), lambda j: (0, j)),
        out_shape=jax.ShapeDtypeStruct((_B, _D), jnp.bfloat16),
        scratch_shapes=[pltpu.VMEM((_B, _AR), jnp.bfloat16)],
        compiler_params=pltpu.CompilerParams(
            allow_input_fusion=[False, False, False, False, True]),
    )(wids, M, C2d, xids, b2)


def kernel(x, lora_A, lora_B, xids, wids):
    x3 = x.reshape(_B, _D // 2, 2).astype(jnp.bfloat16)
    xe, xo = x3[:, :, 0], x3[:, :, 1]
    a2 = lora_A.reshape(_NA, _D // 2, 2 * _R).astype(jnp.bfloat16)
    lora_Bb = lora_B.reshape(_AR, _D).astype(jnp.bfloat16)
    zeros = jnp.zeros((_B, _AR), jnp.float32)

    cmat = _make_sc_count()(xids, wids, zeros)
    M = _stage1(xe, xo, a2)
    out = _stage2(wids, M, cmat, xids, lora_Bb)
    return out.astype(jnp.float16).reshape(_B, 1, _D)


# speedup vs baseline: 1.5093x; 1.5093x over previous
import functools

import jax
import jax.numpy as jnp
from jax import lax
from jax.experimental import pallas as pl
from jax.experimental.pallas import tpu as pltpu
from jax.experimental.pallas import tpu_sc as plsc

_B, _CB, _R, _NA, _D = 32, 32, 64, 8, 4096
_AR = _NA * _R
_DT = 2048
_NP = _NA // 2


@functools.cache
def _make_sc_count():
    mesh = plsc.VectorSubcoreMesh(core_axis_name="c", subcore_axis_name="s", num_cores=1)
    return functools.partial(
        pl.kernel,
        out_type=jax.ShapeDtypeStruct((_B, _AR), jnp.float32),
        mesh=mesh,
        scratch_types=[
            pltpu.VMEM((_CB * _R,), jnp.int32),
            pltpu.VMEM((_CB,), jnp.int32),
            pltpu.VMEM((_B, _AR), jnp.float32),
        ],
        compiler_params=pltpu.CompilerParams(needs_layout_passes=False),
    )(_sc_count_body)


def _sc_count_body(xids_hbm, wids_hbm, zeros_hbm, out_hbm, xids_v, wids_v, c_v):
    cid = lax.axis_index("c")
    sid = lax.axis_index("s")

    @pl.when(jnp.logical_and(cid == 0, sid == 0))
    def _():
        pltpu.sync_copy(xids_hbm, xids_v)
        pltpu.sync_copy(wids_hbm, wids_v)
        pltpu.sync_copy(zeros_hbm, c_v)
        lane = lax.broadcasted_iota(jnp.int32, (16,), 0)
        ones = jnp.ones((16,), jnp.float32)

        def body(i, carry):
            b = xids_v[pl.ds(i * 16, 16)]
            c = lax.shift_right_logical(i, 2)
            a = plsc.load_gather(wids_v, [jnp.full((16,), c, jnp.int32)])
            col = a * _R + (i & 3) * 16 + lane
            plsc.addupdate_scatter(c_v, [b, col], ones)
            return carry

        lax.fori_loop(0, (_CB * _R) // 16, body, 0)
        pltpu.sync_copy(c_v, out_hbm)


def _mm1_body(x_ref, a_ref, m_ref):
    for t in range(2):
        m_ref[t] = jnp.dot(x_ref[...], a_ref[t],
                           preferred_element_type=jnp.float32)


def _stage1(x2d, a_packed):
    return pl.pallas_call(
        _mm1_body,
        grid=(_NP // 2,),
        in_specs=[
            pl.BlockSpec((_B, _D), lambda p: (0, 0)),
            pl.BlockSpec((2, _D, 2 * _R), lambda p: (p, 0, 0)),
        ],
        out_specs=pl.BlockSpec((2, _B, 2 * _R), lambda p: (p, 0, 0)),
        out_shape=jax.ShapeDtypeStruct((_NP, _B, 2 * _R), jnp.float32),
        compiler_params=pltpu.CompilerParams(
            allow_input_fusion=[False, True]),
    )(x2d, a_packed)


def _mm2_body(wids_ref, m_ref, c_ref, xids_ref, b3_ref, out_ref, w_scr):
    j = pl.program_id(0)

    @pl.when(j == 0)
    def _():
        gs = []
        for a in range(_NA):
            wa = wids_ref[a]
            pair = lax.shift_right_logical(wa, 1)
            mrow = m_ref[pl.ds(pair, 1)][0]
            brow = xids_ref[pl.ds(a * _R, _R)]
            msk = brow[None, :] == lax.broadcasted_iota(
                jnp.int32, (_B, _R), 0)
            sel = jnp.where(msk, 1.0, 0.0)
            sel2 = jnp.concatenate([sel, sel], axis=1)
            red = jnp.sum(mrow * sel2, axis=0, keepdims=True)
            gs.append(jnp.where((wa & 1) == 1, red[:, _R:], red[:, :_R]))
        gfull = jnp.concatenate(gs, axis=1)
        w_scr[...] = (c_ref[...] * gfull).astype(jnp.bfloat16)

    out_ref[...] = (2.0 * jnp.dot(
        w_scr[...], b3_ref[...],
        preferred_element_type=jnp.float32)).astype(jnp.bfloat16)


def _stage2(wids, M, C2d, xids, b2):
    return pl.pallas_call(
        _mm2_body,
        grid=(_D // _DT,),
        in_specs=[
            pl.BlockSpec(memory_space=pltpu.SMEM),
            pl.BlockSpec((_NP, _B, 2 * _R), lambda j: (0, 0, 0)),
            pl.BlockSpec((_B, _AR), lambda j: (0, 0)),
            pl.BlockSpec((_AR,), lambda j: (0,)),
            pl.BlockSpec((_AR, _DT), lambda j: (0, j)),
        ],
        out_specs=pl.BlockSpec((_B, _DT), lambda j: (0, j)),
        out_shape=jax.ShapeDtypeStruct((_B, _D), jnp.bfloat16),
        scratch_shapes=[pltpu.VMEM((_B, _AR), jnp.bfloat16)],
        compiler_params=pltpu.CompilerParams(
            allow_input_fusion=[False, False, False, False, True]),
    )(wids, M, C2d, xids, b2)


def kernel(x, lora_A, lora_B, xids, wids):
    x2d = x.reshape(_B, _D).astype(jnp.bfloat16)
    a_packed = lax.reshape(lora_A.reshape(_NP, 2, _D, _R),
                           (_NP, _D, 2 * _R),
                           dimensions=(0, 2, 1, 3)).astype(jnp.bfloat16)
    lora_Bb = lora_B.reshape(_AR, _D).astype(jnp.bfloat16)
    zeros = jnp.zeros((_B, _AR), jnp.float32)

    cmat = _make_sc_count()(xids, wids, zeros)
    M = _stage1(x2d, a_packed)
    out = _stage2(wids, M, cmat, xids, lora_Bb)
    return out.astype(jnp.float16).reshape(_B, 1, _D)
